# baseline (speedup 1.0000x reference)
import functools

import jax
import jax.numpy as jnp
import numpy as np
from jax.experimental import pallas as pl
from jax.experimental.pallas import tpu as pltpu

_SIZE = 32768
_PAD = 0
_SMOOTH = 0.1
_CONF = 1.0 - _SMOOTH
_EPS = _SMOOTH / (_SIZE - 2)
_K = _CONF * float(np.log(_CONF)) + _SMOOTH * float(np.log(_EPS))

_SPLITS = 16
_BR = 8


def _loss_kernel(n_rows, *refs):
    i = pl.program_id(0)
    ni = pl.num_programs(0)
    tgt_refs = refs[:_SPLITS]
    x_refs = refs[_SPLITS:2 * _SPLITS]
    out_ref = refs[2 * _SPLITS]
    acc_ref = refs[2 * _SPLITS + 1]

    @pl.when(i == 0)
    def _init():
        acc_ref[0] = 0.0
        acc_ref[1] = 0.0

    cols = jax.lax.broadcasted_iota(jnp.int32, (_BR, _SIZE), 1)
    for tgt_ref, x_ref in zip(tgt_refs, x_refs):
        tgt = tgt_ref[0]
        nonpad = tgt != _PAD
        x = x_ref[...]
        w = jnp.where(cols == tgt, _CONF, _EPS)
        w = jnp.where(cols == _PAD, 0.0, w)
        w = jnp.where(nonpad, w, 0.0)
        acc_ref[0] += jnp.sum(w * x)
        acc_ref[1] += jnp.sum(nonpad.astype(jnp.float32))

    @pl.when(i == ni - 1)
    def _finish():
        out_ref[0, 0] = (acc_ref[1] * _K - acc_ref[0]) / n_rows


def kernel(x, target):
    n, size = x.shape
    half = n // _SPLITS
    nr = half // _BR
    tgt = target.astype(jnp.int32).reshape(n // _BR, _BR, 1)
    tgt_parts = [tgt] * _SPLITS
    x_parts = [x] * _SPLITS
    out = pl.pallas_call(
        functools.partial(_loss_kernel, float(n)),
        grid=(nr,),
        in_specs=(
            [pl.BlockSpec((1, _BR, 1), lambda i, k=k: (k * nr + i, 0, 0))
             for k in range(_SPLITS)]
            + [pl.BlockSpec((_BR, size), lambda i, k=k: (k * nr + i, 0))
               for k in range(_SPLITS)]
        ),
        out_specs=pl.BlockSpec(memory_space=pltpu.SMEM),
        out_shape=jax.ShapeDtypeStruct((1, 1), jnp.float32),
        scratch_shapes=[pltpu.SMEM((2,), jnp.float32)],
        compiler_params=pltpu.CompilerParams(vmem_limit_bytes=64 * 1024 * 1024),
    )(*tgt_parts, *x_parts)
    return out[0, 0]


# speedup vs baseline: 1.0298x; 1.0298x over previous
import functools

import jax
import jax.numpy as jnp
import numpy as np
from jax.experimental import pallas as pl
from jax.experimental.pallas import tpu as pltpu

_SIZE = 32768
_PAD = 0
_SMOOTH = 0.1
_CONF = 1.0 - _SMOOTH
_EPS = _SMOOTH / (_SIZE - 2)
_K = _CONF * float(np.log(_CONF)) + _SMOOTH * float(np.log(_EPS))

_SPLITS = 8
_BR = 16


def _loss_kernel(n_rows, *refs):
    i = pl.program_id(0)
    ni = pl.num_programs(0)
    tgt_refs = refs[:_SPLITS]
    x_refs = refs[_SPLITS:2 * _SPLITS]
    out_ref = refs[2 * _SPLITS]
    acc_ref = refs[2 * _SPLITS + 1]

    @pl.when(i == 0)
    def _init():
        acc_ref[0] = 0.0
        acc_ref[1] = 0.0

    cols = jax.lax.broadcasted_iota(jnp.int32, (_BR, _SIZE), 1)
    for tgt_ref, x_ref in zip(tgt_refs, x_refs):
        tgt = tgt_ref[0]
        nonpad = tgt != _PAD
        x = x_ref[...]
        w = jnp.where(cols == tgt, _CONF, _EPS)
        w = jnp.where(cols == _PAD, 0.0, w)
        w = jnp.where(nonpad, w, 0.0)
        acc_ref[0] += jnp.sum(w * x)
        acc_ref[1] += jnp.sum(nonpad.astype(jnp.float32))

    @pl.when(i == ni - 1)
    def _finish():
        out_ref[0, 0] = (acc_ref[1] * _K - acc_ref[0]) / n_rows


def kernel(x, target):
    n, size = x.shape
    half = n // _SPLITS
    nr = half // _BR
    tgt = target.astype(jnp.int32).reshape(n // _BR, _BR, 1)
    tgt_parts = [tgt] * _SPLITS
    x_parts = [x] * _SPLITS
    out = pl.pallas_call(
        functools.partial(_loss_kernel, float(n)),
        grid=(nr,),
        in_specs=(
            [pl.BlockSpec((1, _BR, 1), lambda i, k=k: (i * _SPLITS + k, 0, 0))
             for k in range(_SPLITS)]
            + [pl.BlockSpec((_BR, size), lambda i, k=k: (i * _SPLITS + k, 0))
               for k in range(_SPLITS)]
        ),
        out_specs=pl.BlockSpec(memory_space=pltpu.SMEM),
        out_shape=jax.ShapeDtypeStruct((1, 1), jnp.float32),
        scratch_shapes=[pltpu.SMEM((2,), jnp.float32)],
        compiler_params=pltpu.CompilerParams(vmem_limit_bytes=64 * 1024 * 1024),
    )(*tgt_parts, *x_parts)
    return out[0, 0]
